# baseline (speedup 1.0000x reference)
import functools

import jax
import jax.numpy as jnp
from jax import lax
from jax.experimental import pallas as pl
from jax.experimental.pallas import tpu as pltpu
from jax.experimental.pallas import tpu_sc as plsc

_NC = 2
_NS = 16
_NW = _NC * _NS
_K = 1


@functools.lru_cache(maxsize=None)
def _build_gather(nb, s, d):
    pairs_per_w = nb // _NW // 2
    sp = 56
    s2 = 2 * sp
    mesh = plsc.VectorSubcoreMesh(core_axis_name="c", subcore_axis_name="s")

    @functools.partial(
        pl.kernel,
        out_type=jax.ShapeDtypeStruct((nb, sp, d), jnp.float32),
        mesh=mesh,
        scratch_types=[
            pltpu.VMEM((pairs_per_w, s2), jnp.int32),
            pltpu.VMEM((2, s2, d), jnp.float32),
            pltpu.SemaphoreType.DMA((2,)),
            pltpu.SemaphoreType.DMA((2,)),
        ],
    )
    def gather_kernel(table_hbm, idx_hbm, out_hbm, idx_v, rows_v, gsem, ssem):
        wid = lax.axis_index("s") * _NC + lax.axis_index("c")
        base = wid * pairs_per_w
        pltpu.sync_copy(idx_hbm.at[pl.ds(base, pairs_per_w)], idx_v)
        pltpu.async_copy(table_hbm.at[idx_v.at[0]], rows_v.at[0], gsem.at[0])

        @pl.loop(0, pairs_per_w, step=2)
        def round_(r):
            for sub in range(2):
                c = r + sub
                slot = sub
                other = 1 - sub
                pltpu.make_async_copy(
                    table_hbm.at[idx_v.at[c]], rows_v.at[slot], gsem.at[slot]
                ).wait()

                @pl.when(c + 1 < pairs_per_w)
                def _():
                    @pl.when(c >= 1)
                    def _():
                        for h in range(2):
                            pltpu.make_async_copy(
                                rows_v.at[other].at[pl.ds(h * sp, sp)],
                                out_hbm.at[base],
                                ssem.at[other],
                            ).wait()

                    pltpu.async_copy(
                        table_hbm.at[idx_v.at[c + 1]], rows_v.at[other], gsem.at[other]
                    )

                for h in range(2):
                    pltpu.async_copy(
                        rows_v.at[slot].at[pl.ds(h * sp, sp)],
                        out_hbm.at[2 * (base + c) + h],
                        ssem.at[slot],
                    )

        for slot in range(2):
            for h in range(2):
                pltpu.make_async_copy(
                    rows_v.at[slot].at[pl.ds(h * sp, sp)],
                    out_hbm.at[base],
                    ssem.at[slot],
                ).wait()

    return gather_kernel


def kernel(words_ids, table):
    b, s = words_ids.shape
    v, d = table.shape
    idx = jnp.pad(words_ids.astype(jnp.int32).reshape(b // 2, 2, s),
                  ((0, 0), (0, 0), (0, 56 - s))).reshape(b // 2, 2 * 56)
    return _build_gather(b, s, d)(table, idx)[:, :s, :]


# speedup vs baseline: 7.8949x; 7.8949x over previous
import functools

import jax
import jax.numpy as jnp
from jax import lax
from jax.experimental import pallas as pl
from jax.experimental.pallas import tpu as pltpu
from jax.experimental.pallas import tpu_sc as plsc

_NC = 2
_NS = 16
_NW = _NC * _NS
_K = 1


@functools.lru_cache(maxsize=None)
def _build_gather(nb, s, d):
    pairs_per_w = nb // _NW // 2
    s2 = 2 * s
    mesh = plsc.VectorSubcoreMesh(core_axis_name="c", subcore_axis_name="s")

    @functools.partial(
        pl.kernel,
        out_type=jax.ShapeDtypeStruct((nb, s, d), jnp.float32),
        mesh=mesh,
        scratch_types=[
            pltpu.VMEM((pairs_per_w, s2), jnp.int32),
            pltpu.VMEM((4, s2, d), jnp.float32),
            pltpu.SemaphoreType.DMA((4,)),
            pltpu.SemaphoreType.DMA((4,)),
        ],
    )
    def gather_kernel(table_hbm, idx_hbm, out_hbm, idx_v, rows_v, gsem, ssem):
        wid = lax.axis_index("s") * _NC + lax.axis_index("c")
        base = wid * pairs_per_w
        pltpu.sync_copy(idx_hbm.at[pl.ds(base, pairs_per_w)], idx_v)
        for p in range(3):
            pltpu.async_copy(table_hbm.at[idx_v.at[p]], rows_v.at[p], gsem.at[p])

        @pl.loop(0, pairs_per_w, step=4)
        def round_(r):
            for sub in range(4):
                c = r + sub
                slot = sub
                prev = (sub - 1) % 4
                @pl.when(c + 3 < pairs_per_w)
                def _():
                    @pl.when(c >= 1)
                    def _():
                        for h in range(2):
                            pltpu.make_async_copy(
                                rows_v.at[prev].at[pl.ds(h * s, s)],
                                out_hbm.at[base],
                                ssem.at[prev],
                            ).wait()

                    pltpu.async_copy(
                        table_hbm.at[idx_v.at[c + 3]], rows_v.at[prev], gsem.at[prev]
                    )

                pltpu.make_async_copy(
                    table_hbm.at[idx_v.at[c]], rows_v.at[slot], gsem.at[slot]
                ).wait()

                for h in range(2):
                    pltpu.async_copy(
                        rows_v.at[slot].at[pl.ds(h * s, s)],
                        out_hbm.at[2 * (base + c) + h],
                        ssem.at[slot],
                    )

        for slot in range(4):
            for h in range(2):
                pltpu.make_async_copy(
                    rows_v.at[slot].at[pl.ds(h * s, s)],
                    out_hbm.at[base],
                    ssem.at[slot],
                ).wait()

    return gather_kernel


def kernel(words_ids, table):
    b, s = words_ids.shape
    v, d = table.shape
    nb = b // _K
    idx = words_ids.reshape(_K, nb // 2, 2 * s).astype(jnp.int32)
    return _build_gather(nb, s, d)(table, idx[0])


# speedup vs baseline: 7.9814x; 1.0110x over previous
import functools

import jax
import jax.numpy as jnp
from jax import lax
from jax.experimental import pallas as pl
from jax.experimental.pallas import tpu as pltpu
from jax.experimental.pallas import tpu_sc as plsc

_NC = 2
_NS = 16
_NW = _NC * _NS
_K = 1


@functools.lru_cache(maxsize=None)
def _build_gather(nb, s, d):
    pairs_per_w = nb // _NW // 2
    s2 = 2 * s
    mesh = plsc.VectorSubcoreMesh(core_axis_name="c", subcore_axis_name="s")

    @functools.partial(
        pl.kernel,
        out_type=jax.ShapeDtypeStruct((nb, s, d), jnp.float32),
        mesh=mesh,
        scratch_types=[
            pltpu.VMEM((pairs_per_w, s2), jnp.int32),
            pltpu.VMEM((8, s2, d), jnp.float32),
            pltpu.SemaphoreType.DMA((8,)),
            pltpu.SemaphoreType.DMA((8,)),
        ],
    )
    def gather_kernel(table_hbm, idx_hbm, out_hbm, idx_v, rows_v, gsem, ssem):
        wid = lax.axis_index("s") * _NC + lax.axis_index("c")
        base = wid * pairs_per_w
        pltpu.sync_copy(idx_hbm.at[pl.ds(base, pairs_per_w)], idx_v)
        for p in range(7):
            pltpu.async_copy(table_hbm.at[idx_v.at[p]], rows_v.at[p], gsem.at[p])

        @pl.loop(0, pairs_per_w, step=8)
        def round_(r):
            for sub in range(8):
                c = r + sub
                slot = sub
                prev = (sub - 1) % 8
                @pl.when(c + 7 < pairs_per_w)
                def _():
                    @pl.when(c >= 1)
                    def _():
                        for h in range(2):
                            pltpu.make_async_copy(
                                rows_v.at[prev].at[pl.ds(h * s, s)],
                                out_hbm.at[base],
                                ssem.at[prev],
                            ).wait()

                    pltpu.async_copy(
                        table_hbm.at[idx_v.at[c + 7]], rows_v.at[prev], gsem.at[prev]
                    )

                pltpu.make_async_copy(
                    table_hbm.at[idx_v.at[c]], rows_v.at[slot], gsem.at[slot]
                ).wait()

                for h in range(2):
                    pltpu.async_copy(
                        rows_v.at[slot].at[pl.ds(h * s, s)],
                        out_hbm.at[2 * (base + c) + h],
                        ssem.at[slot],
                    )

        for slot in range(8):
            for h in range(2):
                pltpu.make_async_copy(
                    rows_v.at[slot].at[pl.ds(h * s, s)],
                    out_hbm.at[base],
                    ssem.at[slot],
                ).wait()

    return gather_kernel


def kernel(words_ids, table):
    b, s = words_ids.shape
    v, d = table.shape
    nb = b // _K
    idx = words_ids.reshape(_K, nb // 2, 2 * s).astype(jnp.int32)
    return _build_gather(nb, s, d)(table, idx[0])
